# baseline (speedup 1.0000x reference)
import jax
import jax.numpy as jnp
from jax.experimental import pallas as pl
from jax.experimental.pallas import tpu as pltpu


_BLK = 256


_HALF = _BLK // 2


def _diag_matmul_kernel(bt_ref, *refs):
    a_refs, out_ref = refs[:-1], refs[-1]
    dn = (((2,), (2,)), ((0,), (0,)))
    for j, a_ref in enumerate(a_refs):
        blk = j // 2
        start = blk * _BLK + (j % 2) * _HALF
        out_ref[:, :, start : start + _HALF] = jax.lax.dot_general(
            bt_ref[:, :, blk * _BLK : (blk + 1) * _BLK], a_ref[...],
            dimension_numbers=dn, preferred_element_type=jnp.float32,
        )


def kernel(a, b):
    B, H, S, _ = a.shape
    D = b.shape[-1]
    NH = B * H
    a3 = a.reshape(NH, S, S)
    bt = jnp.swapaxes(b.reshape(NH, S, D), 1, 2)
    bt = pltpu.with_memory_space_constraint(bt, pltpu.MemorySpace.HBM)
    n_blocks = S // _BLK

    STREAMS = 2
    a_specs = [
        pl.BlockSpec(
            (NH, _HALF, _BLK),
            (lambda j: (
                lambda i: (
                    0,
                    2 * (STREAMS * i + j // 2) + (j % 2),
                    STREAMS * i + j // 2,
                )
            ))(j),
        )
        for j in range(2 * STREAMS)
    ]
    out_t = pl.pallas_call(
        _diag_matmul_kernel,
        grid=(n_blocks // STREAMS,),
        in_specs=[
            pl.BlockSpec((NH, D, STREAMS * _BLK), lambda i: (0, 0, i)),
            *a_specs,
        ],
        out_specs=pl.BlockSpec((NH, D, STREAMS * _BLK), lambda i: (0, 0, i)),
        out_shape=jax.ShapeDtypeStruct((NH, D, S), jnp.float32),
        compiler_params=pltpu.CompilerParams(
            dimension_semantics=("arbitrary",),
        ),
    )(bt, *([a3] * (2 * STREAMS)))

    return jnp.swapaxes(out_t, 1, 2).reshape(B, H, S, D)


# speedup vs baseline: 1.0058x; 1.0058x over previous
import jax
import jax.numpy as jnp
from jax.experimental import pallas as pl
from jax.experimental.pallas import tpu as pltpu


_BLK = 256
_STREAMS = 2


def _diag_matmul_kernel(bt_ref, *refs):
    a_refs, out_ref = refs[:-1], refs[-1]
    dn = (((2,), (2,)), ((0,), (0,)))
    for j, a_ref in enumerate(a_refs):
        out_ref[:, :, j * _BLK : (j + 1) * _BLK] = jax.lax.dot_general(
            bt_ref[:, :, j * _BLK : (j + 1) * _BLK], a_ref[...],
            dimension_numbers=dn, preferred_element_type=jnp.float32,
        )


def kernel(a, b):
    B, H, S, _ = a.shape
    D = b.shape[-1]
    NH = B * H
    a3 = a.reshape(NH, S, S)
    bt = jnp.swapaxes(b.reshape(NH, S, D), 1, 2)
    bt = pltpu.with_memory_space_constraint(bt, pltpu.MemorySpace.HBM)
    n_blocks = S // _BLK

    a_specs = [
        pl.BlockSpec(
            (NH, _BLK, _BLK),
            (lambda j: (lambda i: (0, _STREAMS * i + j, _STREAMS * i + j)))(j),
        )
        for j in range(_STREAMS)
    ]
    out_t = pl.pallas_call(
        _diag_matmul_kernel,
        grid=(n_blocks // _STREAMS,),
        in_specs=[
            pl.BlockSpec((NH, D, _STREAMS * _BLK), lambda i: (0, 0, i)),
            *a_specs,
        ],
        out_specs=pl.BlockSpec((NH, D, _STREAMS * _BLK), lambda i: (0, 0, i)),
        out_shape=jax.ShapeDtypeStruct((NH, D, S), jnp.float32),
        compiler_params=pltpu.CompilerParams(
            dimension_semantics=("parallel",),
        ),
    )(bt, *([a3] * _STREAMS))

    return jnp.swapaxes(out_t, 1, 2).reshape(B, H, S, D)
